# Initial kernel scaffold; baseline (speedup 1.0000x reference)
import jax
import jax.numpy as jnp
from jax.experimental import pallas as pl


def kernel(x, weights):
    raise NotImplementedError("write your pallas kernel here")


# speedup vs baseline: 1.7468x; 1.7468x over previous
import functools

import numpy as np
import jax
import jax.numpy as jnp
from jax import lax
from jax.experimental import pallas as pl
from jax.experimental.pallas import tpu as pltpu
from jax.experimental.pallas import tpu_sc as plsc

_IRREPS_SPEC = [(128, 1), (64, 3), (32, 5)]

_N = 160000
_F = 480
_NS = 224
_NWEIGHT = 352
_L = 16
_NGROUPS = _F // _L
_NID = 128 // _L
_NG3 = 192 // _L
_NG5 = 160 // _L

_NUM_CORES = 2
_NUM_SUBCORES = 16


def _scale_map():
    idx = []
    c = 0
    for mul, dim in _IRREPS_SPEC:
        for _ in range(mul):
            idx += [c] * dim
            c += 1
    return np.asarray(idx, np.int32)

_SMAP = _scale_map()

_IDX3 = np.asarray(
    [[(16 * j + l) // 3 - 16 * (j // 3) for l in range(_L)]
     for j in range(_NG3)], np.int32)
_IDX5 = np.asarray(
    [[(16 * j + l) // 5 - 16 * (j // 5) for l in range(_L)]
     for j in range(_NG5)], np.int32)
_GIDX = np.concatenate([_IDX3, _IDX5], axis=0)


def _vgather(vec, idx):
    return jnp.take_along_axis(
        vec, idx, axis=0, mode=lax.GatherScatterMode.PROMISE_IN_BOUNDS)


def _build_kernel():
    nw = _NUM_CORES * _NUM_SUBCORES
    rows_per_w = _N // nw
    R = 40
    nchunk = rows_per_w // R
    nbuf = 2

    mesh = plsc.VectorSubcoreMesh(
        core_axis_name="c", subcore_axis_name="s",
        num_cores=_NUM_CORES, num_subcores=_NUM_SUBCORES)

    @functools.partial(
        pl.kernel,
        mesh=mesh,
        compiler_params=pltpu.CompilerParams(
            use_tc_tiling_on_sc=False, needs_layout_passes=False),
        out_type=jax.ShapeDtypeStruct((_N, _F), jnp.float32),
        scratch_types=[
            [pltpu.VMEM((R, _F), jnp.float32) for _ in range(nbuf)],
            [pltpu.VMEM((R, _NWEIGHT), jnp.float32) for _ in range(nbuf)],
            [pltpu.VMEM((R, _F), jnp.float32) for _ in range(nbuf)],
            pltpu.VMEM((_NG3 + _NG5, _L), jnp.int32),
            [pltpu.SemaphoreType.DMA for _ in range(nbuf)],
            [pltpu.SemaphoreType.DMA for _ in range(nbuf)],
        ],
    )
    def k(x_hbm, w_hbm, idx_hbm, o_hbm, xb, wb, ob, idxb, in_sems, out_sems):
        wid = lax.axis_index("s") * _NUM_CORES + lax.axis_index("c")
        base0 = wid * rows_per_w
        pltpu.sync_copy(idx_hbm, idxb)
        idx3 = [idxb[j] for j in range(_NG3)]
        idx5 = [idxb[_NG3 + j] for j in range(_NG5)]

        def start_in(c, b):
            base = base0 + c * R
            pltpu.async_copy(x_hbm.at[pl.ds(base, R)], xb[b], in_sems[b])
            pltpu.async_copy(w_hbm.at[pl.ds(base, R)], wb[b], in_sems[b])

        def wait_in(b):
            pltpu.make_async_copy(x_hbm.at[pl.ds(0, R)], xb[b],
                                  in_sems[b]).wait()
            pltpu.make_async_copy(w_hbm.at[pl.ds(0, R)], wb[b],
                                  in_sems[b]).wait()

        def start_out(c, b):
            base = base0 + c * R
            pltpu.async_copy(ob[b], o_hbm.at[pl.ds(base, R)], out_sems[b])

        def wait_out(b):
            pltpu.make_async_copy(ob[b], o_hbm.at[pl.ds(0, R)],
                                  out_sems[b]).wait()

        def compute(b):
            xv = xb[b]
            wv = wb[b]
            ov = ob[b]

            def row(r, rcarry):
                for g in range(_NID):
                    o = (xv[r, pl.ds(g * _L, _L)] * wv[r, pl.ds(g * _L, _L)]
                         + wv[r, pl.ds(_NS + g * _L, _L)])
                    ov[r, pl.ds(g * _L, _L)] = o
                s3 = [wv[r, pl.ds(128 + k * _L, _L)] for k in range(4)]
                for j in range(_NG3):
                    sv = _vgather(s3[j // 3], idx3[j])
                    c0 = 128 + j * _L
                    ov[r, pl.ds(c0, _L)] = xv[r, pl.ds(c0, _L)] * sv
                s5 = [wv[r, pl.ds(192 + k * _L, _L)] for k in range(2)]
                for j in range(_NG5):
                    sv = _vgather(s5[j // 5], idx5[j])
                    c0 = 320 + j * _L
                    ov[r, pl.ds(c0, _L)] = xv[r, pl.ds(c0, _L)] * sv
                return rcarry

            lax.fori_loop(0, R, row, 0, unroll=False)

        for b in range(nbuf):
            start_in(b, b)

        def step(g, carry):
            for b in range(nbuf):
                c = g * nbuf + b
                wait_in(b)
                lax.cond(c >= nbuf, lambda: wait_out(b), lambda: None)
                compute(b)
                start_out(c, b)
                lax.cond(c + nbuf < nchunk,
                         lambda: start_in(c + nbuf, b), lambda: None)
            return carry

        lax.fori_loop(0, nchunk // nbuf, step, 0, unroll=False)
        for b in range(nbuf):
            wait_out(b)

    return k


@functools.cache
def _get_kernel():
    return _build_kernel()


def kernel(x, weights):
    idx = jnp.asarray(_GIDX)
    return _get_kernel()(x, weights, idx)


# speedup vs baseline: 3.1948x; 1.8290x over previous
import functools

import numpy as np
import jax
import jax.numpy as jnp
from jax import lax
from jax.experimental import pallas as pl
from jax.experimental.pallas import tpu as pltpu
from jax.experimental.pallas import tpu_sc as plsc

_IRREPS_SPEC = [(128, 1), (64, 3), (32, 5)]

_N = 160000
_F = 480
_NS = 224
_NWEIGHT = 352
_L = 16
_NGROUPS = _F // _L
_NID = 128 // _L
_NG3 = 192 // _L
_NG5 = 160 // _L

_NUM_CORES = 2
_NUM_SUBCORES = 16


def _scale_map():
    idx = []
    c = 0
    for mul, dim in _IRREPS_SPEC:
        for _ in range(mul):
            idx += [c] * dim
            c += 1
    return np.asarray(idx, np.int32)

_SMAP = _scale_map()

_IDX3 = np.asarray(
    [[(16 * j + l) // 3 - 16 * (j // 3) for l in range(_L)]
     for j in range(_NG3)], np.int32)
_IDX5 = np.asarray(
    [[(16 * j + l) // 5 - 16 * (j // 5) for l in range(_L)]
     for j in range(_NG5)], np.int32)
_GIDX = np.concatenate([_IDX3, _IDX5], axis=0)


def _vgather(vec, idx):
    return jnp.take_along_axis(
        vec, idx, axis=0, mode=lax.GatherScatterMode.PROMISE_IN_BOUNDS)


def _build_kernel():
    nw = _NUM_CORES * _NUM_SUBCORES
    rows_per_w = _N // nw
    R = 40
    nchunk = rows_per_w // R
    nbuf = 2

    mesh = plsc.VectorSubcoreMesh(
        core_axis_name="c", subcore_axis_name="s",
        num_cores=_NUM_CORES, num_subcores=_NUM_SUBCORES)

    @functools.partial(
        pl.kernel,
        mesh=mesh,
        compiler_params=pltpu.CompilerParams(),
        out_type=jax.ShapeDtypeStruct((_N, _F), jnp.float32),
        scratch_types=[
            [pltpu.VMEM((R, _F), jnp.float32) for _ in range(nbuf)],
            [pltpu.VMEM((R, _NWEIGHT), jnp.float32) for _ in range(nbuf)],
            [pltpu.VMEM((R, _F), jnp.float32) for _ in range(nbuf)],
            pltpu.VMEM((_NG3 + _NG5, _L), jnp.int32),
            [pltpu.SemaphoreType.DMA for _ in range(nbuf)],
            [pltpu.SemaphoreType.DMA for _ in range(nbuf)],
        ],
    )
    def k(x_hbm, w_hbm, idx_hbm, o_hbm, xb, wb, ob, idxb, in_sems, out_sems):
        wid = lax.axis_index("s") * _NUM_CORES + lax.axis_index("c")
        base0 = wid * rows_per_w
        pltpu.sync_copy(idx_hbm, idxb)
        idx3 = [idxb[j] for j in range(_NG3)]
        idx5 = [idxb[_NG3 + j] for j in range(_NG5)]

        def start_in(c, b):
            base = base0 + c * R
            pltpu.async_copy(x_hbm.at[pl.ds(base, R)], xb[b], in_sems[b])
            pltpu.async_copy(w_hbm.at[pl.ds(base, R)], wb[b], in_sems[b])

        def wait_in(b):
            pltpu.make_async_copy(x_hbm.at[pl.ds(0, R)], xb[b],
                                  in_sems[b]).wait()
            pltpu.make_async_copy(w_hbm.at[pl.ds(0, R)], wb[b],
                                  in_sems[b]).wait()

        def start_out(c, b):
            base = base0 + c * R
            pltpu.async_copy(ob[b], o_hbm.at[pl.ds(base, R)], out_sems[b])

        def wait_out(b):
            pltpu.make_async_copy(ob[b], o_hbm.at[pl.ds(0, R)],
                                  out_sems[b]).wait()

        def compute(b):
            xv = xb[b]
            wv = wb[b]
            ov = ob[b]

            def row(r, rcarry):
                for g in range(_NID):
                    o = (xv[r, pl.ds(g * _L, _L)] * wv[r, pl.ds(g * _L, _L)]
                         + wv[r, pl.ds(_NS + g * _L, _L)])
                    ov[r, pl.ds(g * _L, _L)] = o
                s3 = [wv[r, pl.ds(128 + k * _L, _L)] for k in range(4)]
                for j in range(_NG3):
                    sv = _vgather(s3[j // 3], idx3[j])
                    c0 = 128 + j * _L
                    ov[r, pl.ds(c0, _L)] = xv[r, pl.ds(c0, _L)] * sv
                s5 = [wv[r, pl.ds(192 + k * _L, _L)] for k in range(2)]
                for j in range(_NG5):
                    sv = _vgather(s5[j // 5], idx5[j])
                    c0 = 320 + j * _L
                    ov[r, pl.ds(c0, _L)] = xv[r, pl.ds(c0, _L)] * sv
                return rcarry

            lax.fori_loop(0, R, row, 0, unroll=False)

        for b in range(nbuf):
            start_in(b, b)

        def step(g, carry):
            for b in range(nbuf):
                c = g * nbuf + b
                wait_in(b)
                lax.cond(c >= nbuf, lambda: wait_out(b), lambda: None)
                compute(b)
                start_out(c, b)
                lax.cond(c + nbuf < nchunk,
                         lambda: start_in(c + nbuf, b), lambda: None)
            return carry

        lax.fori_loop(0, nchunk // nbuf, step, 0, unroll=False)
        for b in range(nchunk % nbuf):
            c = (nchunk // nbuf) * nbuf + b
            wait_in(b)
            wait_out(b)
            compute(b)
            start_out(c, b)
        for b in range(nbuf):
            wait_out(b)

    return k


@functools.cache
def _get_kernel():
    return _build_kernel()


def kernel(x, weights):
    idx = jnp.asarray(_GIDX)
    return _get_kernel()(x, weights, idx)


# speedup vs baseline: 8.4502x; 2.6449x over previous
import functools

import jax
import jax.numpy as jnp
from jax import lax
from jax.experimental import pallas as pl
from jax.experimental.pallas import tpu as pltpu
from jax.experimental.pallas import tpu_sc as plsc

_N = 160000
_F = 480
_NS = 224
_NWEIGHT = 352
_L = 16
_C = 128

_NUM_CORES = 2
_NUM_SUBCORES = 16
_NFG = 4
_NCG = 8
_NTILES = _N // _C
_NBUF = 3
_MAX_ROUNDS = (_NTILES // _NCG + 1 + _NBUF - 1) // _NBUF


def _build_kernel():
    mesh = plsc.VectorSubcoreMesh(
        core_axis_name="c", subcore_axis_name="s",
        num_cores=_NUM_CORES, num_subcores=_NUM_SUBCORES)
    nbuf = _NBUF

    @functools.partial(
        pl.kernel,
        mesh=mesh,
        out_type=jax.ShapeDtypeStruct((_F, _N), jnp.float32),
        scratch_types=[
            [pltpu.VMEM((32, _C), jnp.float32) for _ in range(nbuf)],
            [pltpu.VMEM((48, _C), jnp.float32) for _ in range(nbuf)],
            [pltpu.VMEM((40, _C), jnp.float32) for _ in range(nbuf)],
            [pltpu.VMEM((32, _C), jnp.float32) for _ in range(nbuf)],
            [pltpu.VMEM((32, _C), jnp.float32) for _ in range(nbuf)],
            [pltpu.VMEM((16, _C), jnp.float32) for _ in range(nbuf)],
            [pltpu.VMEM((8, _C), jnp.float32) for _ in range(nbuf)],
            [pltpu.VMEM((32, _C), jnp.float32) for _ in range(nbuf)],
            [pltpu.VMEM((48, _C), jnp.float32) for _ in range(nbuf)],
            [pltpu.VMEM((40, _C), jnp.float32) for _ in range(nbuf)],
            [pltpu.SemaphoreType.DMA for _ in range(nbuf)],
            [pltpu.SemaphoreType.DMA for _ in range(nbuf)],
        ],
    )
    def k(xt, wt, ot, xs, x1, x2, ws, wh, w1, w2, os_, o1, o2,
          in_sems, out_sems):
        wid = lax.axis_index("s") * _NUM_CORES + lax.axis_index("c")
        fg = wid // _NCG
        cg = wid % _NCG
        nw = (_NTILES - cg + _NCG - 1) // _NCG
        r_s = 32 * fg
        r_h = _NS + 32 * fg
        r_1 = 128 + 48 * fg
        s_1 = 128 + 16 * fg
        r_2 = 320 + 40 * fg
        s_2 = 192 + 8 * fg

        def start_in(c, b):
            col = (c * _NCG + cg) * _C
            cs = pl.ds(col, _C)
            pltpu.async_copy(xt.at[pl.ds(r_s, 32), cs], xs[b], in_sems[b])
            pltpu.async_copy(xt.at[pl.ds(r_1, 48), cs], x1[b], in_sems[b])
            pltpu.async_copy(xt.at[pl.ds(r_2, 40), cs], x2[b], in_sems[b])
            pltpu.async_copy(wt.at[pl.ds(r_s, 32), cs], ws[b], in_sems[b])
            pltpu.async_copy(wt.at[pl.ds(r_h, 32), cs], wh[b], in_sems[b])
            pltpu.async_copy(wt.at[pl.ds(s_1, 16), cs], w1[b], in_sems[b])
            pltpu.async_copy(wt.at[pl.ds(s_2, 8), cs], w2[b], in_sems[b])

        def wait_in(b):
            cs = pl.ds(0, _C)
            for ref, buf in ((xt, xs), (xt, x1), (xt, x2), (wt, ws),
                             (wt, wh), (wt, w1), (wt, w2)):
                pltpu.make_async_copy(
                    ref.at[pl.ds(0, buf[b].shape[0]), cs], buf[b],
                    in_sems[b]).wait()

        def start_out(c, b):
            col = (c * _NCG + cg) * _C
            cs = pl.ds(col, _C)
            pltpu.async_copy(os_[b], ot.at[pl.ds(r_s, 32), cs], out_sems[b])
            pltpu.async_copy(o1[b], ot.at[pl.ds(r_1, 48), cs], out_sems[b])
            pltpu.async_copy(o2[b], ot.at[pl.ds(r_2, 40), cs], out_sems[b])

        def wait_out(b):
            cs = pl.ds(0, _C)
            for buf in (os_, o1, o2):
                pltpu.make_async_copy(
                    buf[b], ot.at[pl.ds(0, buf[b].shape[0]), cs],
                    out_sems[b]).wait()

        def compute(b):
            def vbody(v, carry):
                cv = pl.ds(v * _L, _L)
                for i in range(32):
                    os_[b][i, cv] = (xs[b][i, cv] * ws[b][i, cv]
                                     + wh[b][i, cv])
                for i in range(48):
                    o1[b][i, cv] = x1[b][i, cv] * w1[b][i // 3, cv]
                for i in range(40):
                    o2[b][i, cv] = x2[b][i, cv] * w2[b][i // 5, cv]
                return carry

            lax.fori_loop(0, _C // _L, vbody, 0, unroll=False)

        for b in range(nbuf):
            start_in(b, b)

        def step(g, carry):
            for b in range(nbuf):
                c = g * nbuf + b

                @pl.when(c < nw)
                def _():
                    wait_in(b)

                    @pl.when(c >= nbuf)
                    def _():
                        wait_out(b)

                    compute(b)
                    start_out(c, b)

                    @pl.when(c + nbuf < nw)
                    def _():
                        start_in(c + nbuf, b)
            return carry

        lax.fori_loop(0, _MAX_ROUNDS, step, 0, unroll=False)
        for b in range(nbuf):
            wait_out(b)

    return k


@functools.cache
def _get_kernel():
    return _build_kernel()


def kernel(x, weights):
    out_t = _get_kernel()(x.T, weights.T)
    return out_t.T
